# Initial kernel scaffold; baseline (speedup 1.0000x reference)
import jax
import jax.numpy as jnp
from jax.experimental import pallas as pl


def kernel(input_ids, token_type_ids, token_table, position_table, type_table, a_2, b_2):
    raise NotImplementedError("write your pallas kernel here")


# speedup vs baseline: 2.6407x; 2.6407x over previous
import functools

import jax
import jax.numpy as jnp
from jax import lax
from jax.experimental import pallas as pl
from jax.experimental.pallas import tpu as pltpu
from jax.experimental.pallas import tpu_sc as plsc

LANES = 16


def _ln_body(tok_hbm, ids_hbm, tids_hbm, pos_hbm, type_hbm, ab_hbm, out_hbm,
             idsw_v, tidw_v, pos_v, type_v, ab_v, buf_v, sem,
             *, rows_per_w, S, E, SUB, NC):
    nchunk = E // LANES
    nsub = S // SUB
    wid = lax.axis_index("s") * NC + lax.axis_index("c")
    base = wid * rows_per_w
    ntok = rows_per_w * S

    pltpu.sync_copy(ids_hbm.at[pl.ds(base * S, ntok)], idsw_v)
    pltpu.sync_copy(tids_hbm.at[pl.ds(base * S, ntok)],
                    tidw_v.at[pl.ds(0, ntok)])
    pltpu.sync_copy(pos_hbm.at[pl.ds(0, S)], pos_v)
    pltpu.sync_copy(type_hbm, type_v)
    pltpu.sync_copy(ab_hbm, ab_v)

    for c in range(nchunk):
        sl = pl.ds(c * LANES, LANES)
        type_v[1, sl] = type_v[1, sl] - type_v[0, sl]

    def fold(s, carry):
        for c in range(nchunk):
            sl = pl.ds(c * LANES, LANES)
            pos_v[s, sl] = pos_v[s, sl] + type_v[0, sl]
        return carry

    lax.fori_loop(0, S, fold, 0)

    inv_e = 1.0 / E
    inv_em1 = 1.0 / (E - 1)

    def do_row(r, carry):
        row = base + r
        roff = r * S
        copies = [
            pltpu.async_copy(
                tok_hbm.at[idsw_v.at[pl.ds(roff + j * SUB, SUB)]],
                buf_v.at[pl.ds(j * SUB, SUB)], sem)
            for j in range(nsub)
        ]
        for cp in copies:
            cp.wait()

        def do_tok(t, carry2):
            tidf = tidw_v[pl.ds(roff + t, LANES)][0].astype(jnp.float32)
            xs = []
            acc = None
            accq = None
            for c in range(nchunk):
                sl = pl.ds(c * LANES, LANES)
                x = buf_v[t, sl] + (pos_v[t, sl] + tidf * type_v[1, sl])
                xs.append(x)
                acc = x if acc is None else acc + x
                accq = x * x if accq is None else accq + x * x
            tot = jnp.sum(acc)
            totq = jnp.sum(accq)
            mean = tot * inv_e
            var = (totq - tot * mean) * inv_em1
            i32 = lax.bitcast_convert_type(var, jnp.int32)
            y = lax.bitcast_convert_type(0x5F3759DF - (i32 >> 1), jnp.float32)
            half_v = 0.5 * var
            for _ in range(3):
                y = y * (1.5 - half_v * y * y)
            std = var * y
            d = std + 1e-6
            di = lax.bitcast_convert_type(d, jnp.int32)
            inv = lax.bitcast_convert_type(0x7EF311C3 - di, jnp.float32)
            for _ in range(3):
                inv = inv * (2.0 - d * inv)
            for c in range(nchunk):
                sl = pl.ds(c * LANES, LANES)
                buf_v[t, sl] = (xs[c] - mean) * inv * ab_v[0, sl] + ab_v[1, sl]
            return carry2

        lax.fori_loop(0, S, do_tok, 0)
        pltpu.sync_copy(buf_v, out_hbm.at[row])
        return carry

    lax.fori_loop(0, rows_per_w, do_row, 0)


def kernel(input_ids, token_type_ids, token_table, position_table, type_table,
           a_2, b_2):
    B, S = input_ids.shape
    V, E = token_table.shape
    info = plsc.get_sparse_core_info()
    NC, NS = info.num_cores, info.num_subcores
    NW = NC * NS
    assert B % NW == 0 and E % LANES == 0
    rows_per_w = B // NW
    SUB = max(d for d in range(8, 129, 8) if S % d == 0)

    ids = input_ids.astype(jnp.int32).reshape(B * S)
    tids = token_type_ids.astype(jnp.int32).reshape(B * S)
    ab = jnp.stack([a_2, b_2]).astype(jnp.float32)

    mesh = plsc.VectorSubcoreMesh(core_axis_name="c", subcore_axis_name="s")
    fn = functools.partial(_ln_body, rows_per_w=rows_per_w, S=S, E=E, SUB=SUB,
                           NC=NC)
    out = pl.kernel(
        fn,
        out_type=jax.ShapeDtypeStruct((B, S, E), jnp.float32),
        mesh=mesh,
        compiler_params=pltpu.CompilerParams(needs_layout_passes=False),
        scratch_types=[
            pltpu.VMEM((rows_per_w * S,), jnp.int32),
            pltpu.VMEM((rows_per_w * S + LANES,), jnp.int32),
            pltpu.VMEM((S, E), jnp.float32),
            pltpu.VMEM((2, E), jnp.float32),
            pltpu.VMEM((2, E), jnp.float32),
            pltpu.VMEM((S, E), jnp.float32),
            pltpu.SemaphoreType.DMA,
        ],
    )(token_table, ids, tids, position_table, type_table, ab)
    return out


# speedup vs baseline: 8.2622x; 3.1287x over previous
import functools

import jax
import jax.numpy as jnp
from jax import lax
from jax.experimental import pallas as pl
from jax.experimental.pallas import tpu as pltpu
from jax.experimental.pallas import tpu_sc as plsc

LANES = 16


def _ln_body(tok_hbm, ids_hbm, tids_hbm, pos_hbm, type_hbm, ab_hbm, out_hbm,
             idsw_v, tidw_v, pos_v, type_v, ab_v, buf_v, sem,
             *, rows_per_w, S, E, SUB, NC):
    nchunk = E // LANES
    nsub = S // SUB
    wid = lax.axis_index("s") * NC + lax.axis_index("c")
    base = wid * rows_per_w
    ntok = rows_per_w * S

    pltpu.sync_copy(ids_hbm.at[pl.ds(base * S, ntok)], idsw_v)
    pltpu.sync_copy(tids_hbm.at[pl.ds(base * S, ntok)],
                    tidw_v.at[pl.ds(0, ntok)])
    pltpu.sync_copy(pos_hbm.at[pl.ds(0, S)], pos_v)
    pltpu.sync_copy(type_hbm, type_v)
    pltpu.sync_copy(ab_hbm, ab_v)

    for c in range(nchunk):
        sl = pl.ds(c * LANES, LANES)
        type_v[1, sl] = type_v[1, sl] - type_v[0, sl]

    def fold(s, carry):
        for c in range(nchunk):
            sl = pl.ds(c * LANES, LANES)
            pos_v[s, sl] = pos_v[s, sl] + type_v[0, sl]
        return carry

    lax.fori_loop(0, S, fold, 0)

    inv_e = 1.0 / E
    inv_em1 = 1.0 / (E - 1)

    def do_row(r, carry):
        row = base + r
        roff = r * S
        copies = [
            pltpu.async_copy(
                tok_hbm.at[idsw_v.at[pl.ds(roff + j * SUB, SUB)]],
                buf_v.at[pl.ds(j * SUB, SUB)], sem)
            for j in range(nsub)
        ]
        for cp in copies:
            cp.wait()

        @plsc.parallel_loop(0, S, unroll=2)
        def do_tok(t):
            tidf = tidw_v[pl.ds(roff + t, LANES)][0].astype(jnp.float32)
            xs = []
            acc = None
            accq = None
            for c in range(nchunk):
                sl = pl.ds(c * LANES, LANES)
                x = buf_v[t, sl] + (pos_v[t, sl] + tidf * type_v[1, sl])
                xs.append(x)
                acc = x if acc is None else acc + x
                accq = x * x if accq is None else accq + x * x
            tot = jnp.sum(acc)
            totq = jnp.sum(accq)
            mean = tot * inv_e
            var = (totq - tot * mean) * inv_em1
            i32 = lax.bitcast_convert_type(var, jnp.int32)
            y = lax.bitcast_convert_type(0x5F3759DF - (i32 >> 1), jnp.float32)
            half_v = 0.5 * var
            for _ in range(2):
                y = y * (1.5 - half_v * y * y)
            std = var * y
            d = std + 1e-6
            di = lax.bitcast_convert_type(d, jnp.int32)
            inv = lax.bitcast_convert_type(0x7EF311C3 - di, jnp.float32)
            for _ in range(2):
                inv = inv * (2.0 - d * inv)
            for c in range(nchunk):
                sl = pl.ds(c * LANES, LANES)
                buf_v[t, sl] = (xs[c] - mean) * inv * ab_v[0, sl] + ab_v[1, sl]

        pltpu.sync_copy(buf_v, out_hbm.at[row])
        return carry

    lax.fori_loop(0, rows_per_w, do_row, 0)


def kernel(input_ids, token_type_ids, token_table, position_table, type_table,
           a_2, b_2):
    B, S = input_ids.shape
    V, E = token_table.shape
    info = plsc.get_sparse_core_info()
    NC, NS = info.num_cores, info.num_subcores
    NW = NC * NS
    assert B % NW == 0 and E % LANES == 0
    rows_per_w = B // NW
    SUB = max(d for d in range(8, 129, 8) if S % d == 0)

    ids = input_ids.astype(jnp.int32).reshape(B * S)
    tids = token_type_ids.astype(jnp.int32).reshape(B * S)
    ab = jnp.stack([a_2, b_2]).astype(jnp.float32)

    mesh = plsc.VectorSubcoreMesh(core_axis_name="c", subcore_axis_name="s")
    fn = functools.partial(_ln_body, rows_per_w=rows_per_w, S=S, E=E, SUB=SUB,
                           NC=NC)
    out = pl.kernel(
        fn,
        out_type=jax.ShapeDtypeStruct((B, S, E), jnp.float32),
        mesh=mesh,
        compiler_params=pltpu.CompilerParams(needs_layout_passes=False),
        scratch_types=[
            pltpu.VMEM((rows_per_w * S,), jnp.int32),
            pltpu.VMEM((rows_per_w * S + LANES,), jnp.int32),
            pltpu.VMEM((S, E), jnp.float32),
            pltpu.VMEM((2, E), jnp.float32),
            pltpu.VMEM((2, E), jnp.float32),
            pltpu.VMEM((S, E), jnp.float32),
            pltpu.SemaphoreType.DMA,
        ],
    )(token_table, ids, tids, position_table, type_table, ab)
    return out


# speedup vs baseline: 11.6704x; 1.4125x over previous
import functools

import jax
import jax.numpy as jnp
from jax import lax
from jax.experimental import pallas as pl
from jax.experimental.pallas import tpu as pltpu
from jax.experimental.pallas import tpu_sc as plsc

LANES = 16
NBUF = 3


def _ln_body(tok_hbm, ids_hbm, tids_hbm, pos_hbm, type_hbm, ab_hbm, out_hbm,
             idsw_v, tidw_v, pos_v, type_v, ab_v, buf_v, gsems, ssems,
             *, rows_per_w, S, E, SUB, NC):
    nchunk = E // LANES
    nsub = S // SUB
    wid = lax.axis_index("s") * NC + lax.axis_index("c")
    base = wid * rows_per_w
    ntok = rows_per_w * S

    pltpu.sync_copy(ids_hbm.at[pl.ds(base * S, ntok)], idsw_v)
    pltpu.sync_copy(tids_hbm.at[pl.ds(base * S, ntok)],
                    tidw_v.at[pl.ds(0, ntok)])
    pltpu.sync_copy(pos_hbm.at[pl.ds(0, S)], pos_v)
    pltpu.sync_copy(type_hbm, type_v)
    pltpu.sync_copy(ab_hbm, ab_v)

    for c in range(nchunk):
        sl = pl.ds(c * LANES, LANES)
        type_v[1, sl] = type_v[1, sl] - type_v[0, sl]

    def fold(s, carry):
        for c in range(nchunk):
            sl = pl.ds(c * LANES, LANES)
            pos_v[s, sl] = pos_v[s, sl] + type_v[0, sl]
        return carry

    lax.fori_loop(0, S, fold, 0)

    inv_e = 1.0 / E
    inv_em1 = 1.0 / (E - 1)
    bufs = [buf_v.at[i] for i in range(NBUF)]

    def issue_g(local_row, b):
        roff = local_row * S
        for j in range(nsub):
            pltpu.async_copy(
                tok_hbm.at[idsw_v.at[pl.ds(roff + j * SUB, SUB)]],
                bufs[b].at[pl.ds(j * SUB, SUB)], gsems[b])

    def drain_g(b):
        pltpu.make_async_copy(out_hbm.at[base], bufs[b], gsems[b]).wait()

    def issue_s(local_row, b):
        pltpu.async_copy(bufs[b], out_hbm.at[base + local_row], ssems[b])

    def drain_s(b):
        pltpu.make_async_copy(bufs[b], out_hbm.at[base], ssems[b]).wait()

    def compute(local_row, b):
        roff = local_row * S
        buf = bufs[b]

        @plsc.parallel_loop(0, S, unroll=2)
        def do_tok(t):
            tidf = tidw_v[pl.ds(roff + t, LANES)][0].astype(jnp.float32)
            xs = []
            acc = None
            accq = None
            for c in range(nchunk):
                sl = pl.ds(c * LANES, LANES)
                x = buf[t, sl] + (pos_v[t, sl] + tidf * type_v[1, sl])
                xs.append(x)
                acc = x if acc is None else acc + x
                accq = x * x if accq is None else accq + x * x
            tot = jnp.sum(acc)
            totq = jnp.sum(accq)
            mean = tot * inv_e
            var = (totq - tot * mean) * inv_em1
            i32 = lax.bitcast_convert_type(var, jnp.int32)
            y = lax.bitcast_convert_type(0x5F3759DF - (i32 >> 1), jnp.float32)
            half_v = 0.5 * var
            for _ in range(2):
                y = y * (1.5 - half_v * y * y)
            std = var * y
            d = std + 1e-6
            di = lax.bitcast_convert_type(d, jnp.int32)
            inv = lax.bitcast_convert_type(0x7EF311C3 - di, jnp.float32)
            for _ in range(2):
                inv = inv * (2.0 - d * inv)
            for c in range(nchunk):
                sl = pl.ds(c * LANES, LANES)
                buf[t, sl] = (xs[c] - mean) * inv * ab_v[0, sl] + ab_v[1, sl]

    ntriples = rows_per_w // NBUF
    tail = rows_per_w - ntriples * NBUF

    issue_g(0, 0)

    def triple(r3, carry):
        r0 = r3 * NBUF
        for k in range(NBUF):
            r = r0 + k
            b = k
            nb = (k + 1) % NBUF
            drain_g(b)
            if k < NBUF - 1:
                @pl.when((r3 > 0) | (k >= 2))
                def _():
                    drain_s(nb)
            else:
                drain_s(nb)
            issue_g(r + 1, nb)
            compute(r, b)
            issue_s(r, b)
        return carry

    lax.fori_loop(0, ntriples, triple, 0)

    for k in range(tail):
        r = ntriples * NBUF + k
        b = r % NBUF
        nb = (r + 1) % NBUF
        drain_g(b)
        if k < tail - 1:
            drain_s(nb)
            issue_g(r + 1, nb)
        compute(r, b)
        issue_s(r, b)
    for b in range(NBUF):
        drain_s(b)


def kernel(input_ids, token_type_ids, token_table, position_table, type_table,
           a_2, b_2):
    B, S = input_ids.shape
    V, E = token_table.shape
    info = plsc.get_sparse_core_info()
    NC, NS = info.num_cores, info.num_subcores
    NW = NC * NS
    assert B % NW == 0 and E % LANES == 0
    rows_per_w = B // NW
    assert rows_per_w > 2 * NBUF
    SUB = max(d for d in range(8, 129, 8) if S % d == 0)

    ids = input_ids.astype(jnp.int32).reshape(B * S)
    tids = token_type_ids.astype(jnp.int32).reshape(B * S)
    ab = jnp.stack([a_2, b_2]).astype(jnp.float32)

    mesh = plsc.VectorSubcoreMesh(core_axis_name="c", subcore_axis_name="s")
    fn = functools.partial(_ln_body, rows_per_w=rows_per_w, S=S, E=E, SUB=SUB,
                           NC=NC)
    out = pl.kernel(
        fn,
        out_type=jax.ShapeDtypeStruct((B, S, E), jnp.float32),
        mesh=mesh,
        compiler_params=pltpu.CompilerParams(needs_layout_passes=False),
        scratch_types=[
            pltpu.VMEM((rows_per_w * S,), jnp.int32),
            pltpu.VMEM((rows_per_w * S + LANES,), jnp.int32),
            pltpu.VMEM((S, E), jnp.float32),
            pltpu.VMEM((2, E), jnp.float32),
            pltpu.VMEM((2, E), jnp.float32),
            pltpu.VMEM((NBUF, S, E), jnp.float32),
            [pltpu.SemaphoreType.DMA] * NBUF,
            [pltpu.SemaphoreType.DMA] * NBUF,
        ],
    )(token_table, ids, tids, position_table, type_table, ab)
    return out
